# baseline (speedup 1.0000x reference)
import functools

import jax
import jax.numpy as jnp
from jax import lax
from jax.experimental import pallas as pl
from jax.experimental.pallas import tpu as pltpu
from jax.experimental.pallas import tpu_sc as plsc

N = 10000
E = 320000
EP = 327680
NTILES = 16
TILE_CHUNKS = EP // (NTILES * 128)
DEG_CHUNKS = EP // (32 * 128)
ROWS_A = 624
ROWS_TAIL = N - ROWS_A * NTILES
ZR = 208


def _mesh():
    return plsc.VectorSubcoreMesh(
        core_axis_name="c", subcore_axis_name="s", num_cores=2, num_subcores=16
    )


@functools.cache
def _make_deg():
    @functools.partial(
        pl.kernel,
        out_type=jax.ShapeDtypeStruct((2 * N,), jnp.float32),
        mesh=_mesh(),
        scratch_types=[
            pltpu.VMEM((DEG_CHUNKS, 128), jnp.int32),
            pltpu.VMEM((128,), jnp.float32),
            pltpu.VMEM((ZR,), jnp.float32),
            pltpu.VMEM_SHARED((N + 8,), jnp.float32),
        ],
    )
    def deg_kernel(dst4_hbm, out_hbm, dst_v, ones_v, stag_v, acc_sh):
        c = lax.axis_index("c")
        s = lax.axis_index("s")
        w = c * NTILES + s
        base = s * ROWS_A
        for i in range(ZR // 16):
            stag_v[pl.ds(i * 16, 16)] = jnp.zeros((16,), jnp.float32)
        for k in range(3):
            pltpu.sync_copy(stag_v, acc_sh.at[pl.ds(base + k * ZR, ZR)])

        @pl.when(s == NTILES - 1)
        def _():
            pltpu.sync_copy(stag_v.at[pl.ds(0, 24)], acc_sh.at[pl.ds(N - ROWS_TAIL, 24)])

        for i in range(8):
            ones_v[pl.ds(i * 16, 16)] = jnp.ones((16,), jnp.float32)
        pltpu.sync_copy(dst4_hbm.at[w], dst_v)
        plsc.subcore_barrier()

        def body(j, carry):
            pltpu.sync_copy(ones_v, acc_sh.at[dst_v.at[j]], add=True)
            return carry

        lax.fori_loop(0, DEG_CHUNKS, body, 0)
        plsc.subcore_barrier()
        for k in range(3):
            pltpu.sync_copy(acc_sh.at[pl.ds(base + k * ZR, ZR)], stag_v)
            pltpu.sync_copy(stag_v, out_hbm.at[pl.ds(c * N + base + k * ZR, ZR)])

        @pl.when(s == NTILES - 1)
        def _():
            pltpu.sync_copy(
                acc_sh.at[pl.ds(N - ROWS_TAIL, ROWS_TAIL)], stag_v.at[pl.ds(0, ROWS_TAIL)]
            )
            pltpu.sync_copy(
                stag_v.at[pl.ds(0, ROWS_TAIL)],
                out_hbm.at[pl.ds(c * N + N - ROWS_TAIL, ROWS_TAIL)],
            )

    return deg_kernel


@functools.cache
def _make_agg_ns():
    NH = N // 2
    RA = 312
    ZC = 104

    @functools.partial(
        pl.kernel,
        out_type=jax.ShapeDtypeStruct((2, NH, 128), jnp.float32),
        mesh=_mesh(),
        scratch_types=[
            pltpu.VMEM((TILE_CHUNKS, 128), jnp.int32),
            pltpu.VMEM((TILE_CHUNKS, 128), jnp.int32),
            pltpu.VMEM((2, 128, 128), jnp.float32),
            pltpu.VMEM((ZC, 128), jnp.float32),
            pltpu.VMEM_SHARED((NH + 8, 128), jnp.float32),
            [pltpu.SemaphoreType.DMA] * 2,
            [pltpu.SemaphoreType.DMA] * 2,
        ],
    )
    def agg(g_hbm, src3_hbm, dstns_hbm, out_hbm, src_v, dst_v, rows_v, stag_v, acc_sh, semG, semS):
        c = lax.axis_index("c")
        s = lax.axis_index("s")
        base = s * RA

        def zrow(i, carry):
            for f in range(8):
                stag_v[i, pl.ds(f * 16, 16)] = jnp.zeros((16,), jnp.float32)
            return carry

        lax.fori_loop(0, ZC, zrow, 0)
        for k in range(3):
            pltpu.sync_copy(stag_v, acc_sh.at[pl.ds(base + k * ZC, ZC)])

        @pl.when(s == NTILES - 1)
        def _():
            pltpu.sync_copy(stag_v.at[pl.ds(0, 16)], acc_sh.at[pl.ds(NH - 8, 16)])

        pltpu.sync_copy(src3_hbm.at[s], src_v)
        pltpu.sync_copy(dstns_hbm.at[c, s], dst_v)
        plsc.subcore_barrier()

        for k in range(2):
            pltpu.async_copy(g_hbm.at[src_v.at[k]], rows_v.at[k], semG[k])

        def body(jj, carry):
            j0 = jj * 2
            for k in range(2):
                pltpu.make_async_copy(g_hbm.at[pl.ds(0, 128)], rows_v.at[k], semG[k]).wait()
                pltpu.async_copy(rows_v.at[k], acc_sh.at[dst_v.at[j0 + k]], semS[k], add=True)
            for k in range(2):
                pltpu.make_async_copy(g_hbm.at[pl.ds(0, 128)], rows_v.at[k], semS[k]).wait()

                @pl.when(jj < TILE_CHUNKS // 2 - 1)
                def _():
                    pltpu.async_copy(g_hbm.at[src_v.at[j0 + k + 2]], rows_v.at[k], semG[k])

            return carry

        lax.fori_loop(0, TILE_CHUNKS // 2, body, 0)
        plsc.subcore_barrier()
        for k in range(3):
            pltpu.sync_copy(acc_sh.at[pl.ds(base + k * ZC, ZC)], stag_v)
            pltpu.sync_copy(stag_v, out_hbm.at[c, pl.ds(base + k * ZC, ZC)])

        @pl.when(s == NTILES - 1)
        def _():
            pltpu.sync_copy(acc_sh.at[pl.ds(NH - 8, 8)], stag_v.at[pl.ds(0, 8)])
            pltpu.sync_copy(stag_v.at[pl.ds(0, 8)], out_hbm.at[c, pl.ds(NH - 8, 8)])

    return agg


_B = 1000


def _tc_prep(deg2, x):
    def body(deg_ref, x_ref, d_ref, g_ref):
        deg = deg_ref[0] + deg_ref[1] + 1.0
        d = lax.rsqrt(deg)
        d_ref[...] = d
        g_ref[...] = x_ref[...] * d

    return pl.pallas_call(
        body,
        grid=(N // _B,),
        in_specs=[
            pl.BlockSpec((2, _B, 1), lambda i: (0, i, 0)),
            pl.BlockSpec((_B, 128), lambda i: (i, 0)),
        ],
        out_specs=[
            pl.BlockSpec((_B, 1), lambda i: (i, 0)),
            pl.BlockSpec((_B, 128), lambda i: (i, 0)),
        ],
        out_shape=[
            jax.ShapeDtypeStruct((N, 1), jnp.float32),
            jax.ShapeDtypeStruct((N, 128), jnp.float32),
        ],
    )(deg2.reshape(2, N, 1), x)


def _tc_layer1(acc1, g1, d, W1, b1):
    def body(a_ref, g_ref, d_ref, W_ref, b_ref, out_ref):
        y = (a_ref[0] + g_ref[...]) * d_ref[...]
        h = jnp.dot(y, W_ref[...], preferred_element_type=jnp.float32) + b_ref[...]
        g = jnp.maximum(h, 0.0) * d_ref[...]
        out_ref[0] = g[:, :128]
        out_ref[1] = g[:, 128:]

    return pl.pallas_call(
        body,
        grid=(N // _B,),
        in_specs=[
            pl.BlockSpec((1, _B, 128), lambda i: (i // 5, i % 5, 0)),
            pl.BlockSpec((_B, 128), lambda i: (i, 0)),
            pl.BlockSpec((_B, 1), lambda i: (i, 0)),
            pl.BlockSpec((128, 256), lambda i: (0, 0)),
            pl.BlockSpec((1, 256), lambda i: (0, 0)),
        ],
        out_specs=pl.BlockSpec((2, _B, 128), lambda i: (0, i, 0)),
        out_shape=jax.ShapeDtypeStruct((2, N, 128), jnp.float32),
    )(acc1, g1, d, W1, b1.reshape(1, 256))


def _tc_layer2(acc2a, acc2b, g2, d, W2, b2, fc_W, fc_b):
    def body(a0_ref, a1_ref, g_ref, d_ref, W_ref, b_ref, fw_ref, fb_ref, out_ref):
        a = jnp.concatenate([a0_ref[0] + g_ref[0], a1_ref[0] + g_ref[1]], axis=1)
        y = a * d_ref[...]
        h = jnp.dot(y, W_ref[...], preferred_element_type=jnp.float32) + b_ref[...]
        h = jnp.maximum(h, 0.0)
        logits = jnp.dot(h, fw_ref[...], preferred_element_type=jnp.float32) + fb_ref[...]
        m = jnp.max(logits, axis=1, keepdims=True)
        lse = jnp.log(jnp.sum(jnp.exp(logits - m), axis=1, keepdims=True)) + m
        out_ref[...] = logits - lse

    return pl.pallas_call(
        body,
        grid=(N // _B,),
        in_specs=[
            pl.BlockSpec((1, _B, 128), lambda i: (i // 5, i % 5, 0)),
            pl.BlockSpec((1, _B, 128), lambda i: (i // 5, i % 5, 0)),
            pl.BlockSpec((2, _B, 128), lambda i: (0, i, 0)),
            pl.BlockSpec((_B, 1), lambda i: (i, 0)),
            pl.BlockSpec((256, 256), lambda i: (0, 0)),
            pl.BlockSpec((1, 256), lambda i: (0, 0)),
            pl.BlockSpec((256, 64), lambda i: (0, 0)),
            pl.BlockSpec((1, 64), lambda i: (0, 0)),
        ],
        out_specs=pl.BlockSpec((_B, 64), lambda i: (i, 0)),
        out_shape=jax.ShapeDtypeStruct((N, 64), jnp.float32),
    )(acc2a, acc2b, g2, d, W2, b2.reshape(1, 256), fc_W, fc_b.reshape(1, 64))


def kernel(x, edge_index, W1, b1, W2, b2, fc_W, fc_b):
    src = edge_index[0]
    dst = edge_index[1]
    pad = EP - E
    src_p = jnp.concatenate([src, jnp.zeros((pad,), jnp.int32)])
    dst_p = jnp.concatenate([dst, jnp.full((pad,), N, jnp.int32)])
    dst32 = dst_p.reshape(32, DEG_CHUNKS, 128)
    src3 = src_p.reshape(NTILES, TILE_CHUNKS, 128)
    nh = N // 2
    dstns = jnp.stack([
        jnp.where(dst_p < nh, dst_p, nh),
        jnp.where(dst_p >= nh, dst_p - nh, nh),
    ]).reshape(2, NTILES, TILE_CHUNKS, 128)

    deg2 = _make_deg()(dst32)
    d, g1 = _tc_prep(deg2, x)
    agg = _make_agg_ns()
    acc1 = agg(g1, src3, dstns)
    g2 = _tc_layer1(acc1, g1, d, W1, b1)
    acc2a = agg(g2[0], src3, dstns)
    acc2b = agg(g2[1], src3, dstns)
    return _tc_layer2(acc2a, acc2b, g2, d, W2, b2, fc_W, fc_b)


# speedup vs baseline: 1.0303x; 1.0303x over previous
import functools

import jax
import jax.numpy as jnp
from jax import lax
from jax.experimental import pallas as pl
from jax.experimental.pallas import tpu as pltpu
from jax.experimental.pallas import tpu_sc as plsc

N = 10000
E = 320000
EP = 327680
NTILES = 16
TILE_CHUNKS = EP // (NTILES * 128)
DEG_CHUNKS = EP // (32 * 128)
ROWS_A = 624
ROWS_TAIL = N - ROWS_A * NTILES
ZR = 208


def _mesh():
    return plsc.VectorSubcoreMesh(
        core_axis_name="c", subcore_axis_name="s", num_cores=2, num_subcores=16
    )


@functools.cache
def _make_deg():
    @functools.partial(
        pl.kernel,
        out_type=jax.ShapeDtypeStruct((2 * N,), jnp.float32),
        mesh=_mesh(),
        scratch_types=[
            pltpu.VMEM((DEG_CHUNKS, 128), jnp.int32),
            pltpu.VMEM((128,), jnp.float32),
            pltpu.VMEM((ZR,), jnp.float32),
            pltpu.VMEM_SHARED((N + 8,), jnp.float32),
        ],
    )
    def deg_kernel(dst4_hbm, out_hbm, dst_v, ones_v, stag_v, acc_sh):
        c = lax.axis_index("c")
        s = lax.axis_index("s")
        w = c * NTILES + s
        base = s * ROWS_A
        for i in range(ZR // 16):
            stag_v[pl.ds(i * 16, 16)] = jnp.zeros((16,), jnp.float32)
        for k in range(3):
            pltpu.sync_copy(stag_v, acc_sh.at[pl.ds(base + k * ZR, ZR)])

        @pl.when(s == NTILES - 1)
        def _():
            pltpu.sync_copy(stag_v.at[pl.ds(0, 24)], acc_sh.at[pl.ds(N - ROWS_TAIL, 24)])

        for i in range(8):
            ones_v[pl.ds(i * 16, 16)] = jnp.ones((16,), jnp.float32)
        pltpu.sync_copy(dst4_hbm.at[w], dst_v)
        plsc.subcore_barrier()

        def body(j, carry):
            pltpu.sync_copy(ones_v, acc_sh.at[dst_v.at[j]], add=True)
            return carry

        lax.fori_loop(0, DEG_CHUNKS, body, 0)
        plsc.subcore_barrier()
        for k in range(3):
            pltpu.sync_copy(acc_sh.at[pl.ds(base + k * ZR, ZR)], stag_v)
            pltpu.sync_copy(stag_v, out_hbm.at[pl.ds(c * N + base + k * ZR, ZR)])

        @pl.when(s == NTILES - 1)
        def _():
            pltpu.sync_copy(
                acc_sh.at[pl.ds(N - ROWS_TAIL, ROWS_TAIL)], stag_v.at[pl.ds(0, ROWS_TAIL)]
            )
            pltpu.sync_copy(
                stag_v.at[pl.ds(0, ROWS_TAIL)],
                out_hbm.at[pl.ds(c * N + N - ROWS_TAIL, ROWS_TAIL)],
            )

    return deg_kernel


@functools.cache
def _make_agg_ns():
    NH = N // 2
    RA = 312
    ZC = 104

    @functools.partial(
        pl.kernel,
        out_type=jax.ShapeDtypeStruct((2, NH, 128), jnp.float32),
        mesh=_mesh(),
        scratch_types=[
            pltpu.VMEM((TILE_CHUNKS, 128), jnp.int32),
            pltpu.VMEM((TILE_CHUNKS, 128), jnp.int32),
            pltpu.VMEM((128, 128), jnp.float32),
            pltpu.VMEM((128, 128), jnp.float32),
            pltpu.VMEM((ZC, 128), jnp.float32),
            pltpu.VMEM_SHARED((NH + 8, 128), jnp.float32),
            pltpu.SemaphoreType.DMA,
            pltpu.SemaphoreType.DMA,
        ],
    )
    def agg(g_hbm, src3_hbm, dstns_hbm, out_hbm, src_v, dst_v, rows_v, rows2_v, stag_v, acc_sh, semA, semB):
        c = lax.axis_index("c")
        s = lax.axis_index("s")
        base = s * RA

        def zrow(i, carry):
            for f in range(8):
                stag_v[i, pl.ds(f * 16, 16)] = jnp.zeros((16,), jnp.float32)
            return carry

        lax.fori_loop(0, ZC, zrow, 0)
        for k in range(3):
            pltpu.sync_copy(stag_v, acc_sh.at[pl.ds(base + k * ZC, ZC)])

        @pl.when(s == NTILES - 1)
        def _():
            pltpu.sync_copy(stag_v.at[pl.ds(0, 16)], acc_sh.at[pl.ds(NH - 8, 16)])

        pltpu.sync_copy(src3_hbm.at[s], src_v)
        pltpu.sync_copy(dstns_hbm.at[c, s], dst_v)
        plsc.subcore_barrier()

        pltpu.async_copy(g_hbm.at[src_v.at[0]], rows_v, semA)

        def body(jj, carry):
            j0 = jj * 2
            j1 = j0 + 1
            pltpu.async_copy(g_hbm.at[src_v.at[j1]], rows2_v, semB)
            pltpu.make_async_copy(g_hbm.at[pl.ds(0, 128)], rows_v, semA).wait()
            pltpu.sync_copy(rows_v, acc_sh.at[dst_v.at[j0]], add=True)

            @pl.when(jj < TILE_CHUNKS // 2 - 1)
            def _():
                pltpu.async_copy(g_hbm.at[src_v.at[j0 + 2]], rows_v, semA)

            pltpu.make_async_copy(g_hbm.at[pl.ds(0, 128)], rows2_v, semB).wait()
            pltpu.sync_copy(rows2_v, acc_sh.at[dst_v.at[j1]], add=True)
            return carry

        lax.fori_loop(0, TILE_CHUNKS // 2, body, 0)
        plsc.subcore_barrier()
        for k in range(3):
            pltpu.sync_copy(acc_sh.at[pl.ds(base + k * ZC, ZC)], stag_v)
            pltpu.sync_copy(stag_v, out_hbm.at[c, pl.ds(base + k * ZC, ZC)])

        @pl.when(s == NTILES - 1)
        def _():
            pltpu.sync_copy(acc_sh.at[pl.ds(NH - 8, 8)], stag_v.at[pl.ds(0, 8)])
            pltpu.sync_copy(stag_v.at[pl.ds(0, 8)], out_hbm.at[c, pl.ds(NH - 8, 8)])

    return agg


_B = 1000


def _tc_prep(deg2, x):
    def body(deg_ref, x_ref, d_ref, g_ref):
        deg = deg_ref[0] + deg_ref[1] + 1.0
        d = lax.rsqrt(deg)
        d_ref[...] = d
        g_ref[...] = x_ref[...] * d

    return pl.pallas_call(
        body,
        grid=(N // _B,),
        in_specs=[
            pl.BlockSpec((2, _B, 1), lambda i: (0, i, 0)),
            pl.BlockSpec((_B, 128), lambda i: (i, 0)),
        ],
        out_specs=[
            pl.BlockSpec((_B, 1), lambda i: (i, 0)),
            pl.BlockSpec((_B, 128), lambda i: (i, 0)),
        ],
        out_shape=[
            jax.ShapeDtypeStruct((N, 1), jnp.float32),
            jax.ShapeDtypeStruct((N, 128), jnp.float32),
        ],
    )(deg2.reshape(2, N, 1), x)


def _tc_layer1(acc1, g1, d, W1, b1):
    def body(a_ref, g_ref, d_ref, W_ref, b_ref, out_ref):
        y = (a_ref[0] + g_ref[...]) * d_ref[...]
        h = jnp.dot(y, W_ref[...], preferred_element_type=jnp.float32) + b_ref[...]
        g = jnp.maximum(h, 0.0) * d_ref[...]
        out_ref[0] = g[:, :128]
        out_ref[1] = g[:, 128:]

    return pl.pallas_call(
        body,
        grid=(N // _B,),
        in_specs=[
            pl.BlockSpec((1, _B, 128), lambda i: (i // 5, i % 5, 0)),
            pl.BlockSpec((_B, 128), lambda i: (i, 0)),
            pl.BlockSpec((_B, 1), lambda i: (i, 0)),
            pl.BlockSpec((128, 256), lambda i: (0, 0)),
            pl.BlockSpec((1, 256), lambda i: (0, 0)),
        ],
        out_specs=pl.BlockSpec((2, _B, 128), lambda i: (0, i, 0)),
        out_shape=jax.ShapeDtypeStruct((2, N, 128), jnp.float32),
    )(acc1, g1, d, W1, b1.reshape(1, 256))


def _tc_layer2(acc2a, acc2b, g2, d, W2, b2, fc_W, fc_b):
    def body(a0_ref, a1_ref, g_ref, d_ref, W_ref, b_ref, fw_ref, fb_ref, out_ref):
        a = jnp.concatenate([a0_ref[0] + g_ref[0], a1_ref[0] + g_ref[1]], axis=1)
        y = a * d_ref[...]
        h = jnp.dot(y, W_ref[...], preferred_element_type=jnp.float32) + b_ref[...]
        h = jnp.maximum(h, 0.0)
        logits = jnp.dot(h, fw_ref[...], preferred_element_type=jnp.float32) + fb_ref[...]
        m = jnp.max(logits, axis=1, keepdims=True)
        lse = jnp.log(jnp.sum(jnp.exp(logits - m), axis=1, keepdims=True)) + m
        out_ref[...] = logits - lse

    return pl.pallas_call(
        body,
        grid=(N // _B,),
        in_specs=[
            pl.BlockSpec((1, _B, 128), lambda i: (i // 5, i % 5, 0)),
            pl.BlockSpec((1, _B, 128), lambda i: (i // 5, i % 5, 0)),
            pl.BlockSpec((2, _B, 128), lambda i: (0, i, 0)),
            pl.BlockSpec((_B, 1), lambda i: (i, 0)),
            pl.BlockSpec((256, 256), lambda i: (0, 0)),
            pl.BlockSpec((1, 256), lambda i: (0, 0)),
            pl.BlockSpec((256, 64), lambda i: (0, 0)),
            pl.BlockSpec((1, 64), lambda i: (0, 0)),
        ],
        out_specs=pl.BlockSpec((_B, 64), lambda i: (i, 0)),
        out_shape=jax.ShapeDtypeStruct((N, 64), jnp.float32),
    )(acc2a, acc2b, g2, d, W2, b2.reshape(1, 256), fc_W, fc_b.reshape(1, 64))


def kernel(x, edge_index, W1, b1, W2, b2, fc_W, fc_b):
    src = edge_index[0]
    dst = edge_index[1]
    pad = EP - E
    src_p = jnp.concatenate([src, jnp.zeros((pad,), jnp.int32)])
    dst_p = jnp.concatenate([dst, jnp.full((pad,), N, jnp.int32)])
    dst32 = dst_p.reshape(32, DEG_CHUNKS, 128)
    src3 = src_p.reshape(NTILES, TILE_CHUNKS, 128)
    nh = N // 2
    dstns = jnp.stack([
        jnp.where(dst_p < nh, dst_p, nh),
        jnp.where(dst_p >= nh, dst_p - nh, nh),
    ]).reshape(2, NTILES, TILE_CHUNKS, 128)

    deg2 = _make_deg()(dst32)
    d, g1 = _tc_prep(deg2, x)
    agg = _make_agg_ns()
    acc1 = agg(g1, src3, dstns)
    g2 = _tc_layer1(acc1, g1, d, W1, b1)
    acc2a = agg(g2[0], src3, dstns)
    acc2b = agg(g2[1], src3, dstns)
    return _tc_layer2(acc2a, acc2b, g2, d, W2, b2, fc_W, fc_b)


# speedup vs baseline: 1.0820x; 1.0501x over previous
import functools

import jax
import jax.numpy as jnp
from jax import lax
from jax.experimental import pallas as pl
from jax.experimental.pallas import tpu as pltpu
from jax.experimental.pallas import tpu_sc as plsc

N = 10000
E = 320000
EP = 327680
NTILES = 16
TILE_CHUNKS = EP // (NTILES * 128)
DEG_CHUNKS = EP // (32 * 128)
ROWS_A = 624
ROWS_TAIL = N - ROWS_A * NTILES
ZR = 208


def _mesh():
    return plsc.VectorSubcoreMesh(
        core_axis_name="c", subcore_axis_name="s", num_cores=2, num_subcores=16
    )


@functools.cache
def _make_deg():
    @functools.partial(
        pl.kernel,
        out_type=jax.ShapeDtypeStruct((2 * N,), jnp.float32),
        mesh=_mesh(),
        scratch_types=[
            pltpu.VMEM((DEG_CHUNKS, 128), jnp.int32),
            pltpu.VMEM((128,), jnp.float32),
            pltpu.VMEM((ZR,), jnp.float32),
            pltpu.VMEM_SHARED((N + 8,), jnp.float32),
        ],
    )
    def deg_kernel(dst4_hbm, out_hbm, dst_v, ones_v, stag_v, acc_sh):
        c = lax.axis_index("c")
        s = lax.axis_index("s")
        w = c * NTILES + s
        base = s * ROWS_A
        for i in range(ZR // 16):
            stag_v[pl.ds(i * 16, 16)] = jnp.zeros((16,), jnp.float32)
        for k in range(3):
            pltpu.sync_copy(stag_v, acc_sh.at[pl.ds(base + k * ZR, ZR)])

        @pl.when(s == NTILES - 1)
        def _():
            pltpu.sync_copy(stag_v.at[pl.ds(0, 24)], acc_sh.at[pl.ds(N - ROWS_TAIL, 24)])

        for i in range(8):
            ones_v[pl.ds(i * 16, 16)] = jnp.ones((16,), jnp.float32)
        pltpu.sync_copy(dst4_hbm.at[w], dst_v)
        plsc.subcore_barrier()

        def body(j, carry):
            pltpu.sync_copy(ones_v, acc_sh.at[dst_v.at[j]], add=True)
            return carry

        lax.fori_loop(0, DEG_CHUNKS, body, 0)
        plsc.subcore_barrier()
        for k in range(3):
            pltpu.sync_copy(acc_sh.at[pl.ds(base + k * ZR, ZR)], stag_v)
            pltpu.sync_copy(stag_v, out_hbm.at[pl.ds(c * N + base + k * ZR, ZR)])

        @pl.when(s == NTILES - 1)
        def _():
            pltpu.sync_copy(
                acc_sh.at[pl.ds(N - ROWS_TAIL, ROWS_TAIL)], stag_v.at[pl.ds(0, ROWS_TAIL)]
            )
            pltpu.sync_copy(
                stag_v.at[pl.ds(0, ROWS_TAIL)],
                out_hbm.at[pl.ds(c * N + N - ROWS_TAIL, ROWS_TAIL)],
            )

    return deg_kernel


@functools.cache
def _make_agg_ns():
    NH = N // 2
    RA = 312
    ZC = 104

    @functools.partial(
        pl.kernel,
        out_type=jax.ShapeDtypeStruct((2, NH, 128), jnp.float32),
        mesh=_mesh(),
        scratch_types=[
            pltpu.VMEM((TILE_CHUNKS, 128), jnp.int32),
            pltpu.VMEM((TILE_CHUNKS, 128), jnp.int32),
            pltpu.VMEM((128, 128), jnp.float32),
            pltpu.VMEM((128, 128), jnp.float32),
            pltpu.VMEM((ZC, 128), jnp.float32),
            pltpu.VMEM_SHARED((NH + 128, 128), jnp.float32),
            pltpu.SemaphoreType.DMA,
            pltpu.SemaphoreType.DMA,
        ],
    )
    def agg(g_hbm, src3_hbm, dstns_hbm, out_hbm, src_v, dst_v, rows_v, rows2_v, stag_v, acc_sh, semA, semB):
        c = lax.axis_index("c")
        s = lax.axis_index("s")
        base = s * RA

        def zrow(i, carry):
            for f in range(8):
                stag_v[i, pl.ds(f * 16, 16)] = jnp.zeros((16,), jnp.float32)
            return carry

        lax.fori_loop(0, ZC, zrow, 0)
        for k in range(3):
            pltpu.sync_copy(stag_v, acc_sh.at[pl.ds(base + k * ZC, ZC)])

        @pl.when(s == NTILES - 1)
        def _():
            pltpu.sync_copy(stag_v.at[pl.ds(0, 16)], acc_sh.at[pl.ds(NH - 8, 16)])

        pltpu.sync_copy(src3_hbm.at[s], src_v)
        pltpu.sync_copy(dstns_hbm.at[c, s], dst_v)
        plsc.subcore_barrier()

        pltpu.async_copy(g_hbm.at[src_v.at[0]], rows_v, semA)

        def body(jj, carry):
            j0 = jj * 2
            j1 = j0 + 1
            pltpu.async_copy(g_hbm.at[src_v.at[j1]], rows2_v, semB)
            pltpu.make_async_copy(g_hbm.at[pl.ds(0, 128)], rows_v, semA).wait()
            pltpu.sync_copy(rows_v, acc_sh.at[dst_v.at[j0]], add=True)

            @pl.when(jj < TILE_CHUNKS // 2 - 1)
            def _():
                pltpu.async_copy(g_hbm.at[src_v.at[j0 + 2]], rows_v, semA)

            pltpu.make_async_copy(g_hbm.at[pl.ds(0, 128)], rows2_v, semB).wait()
            pltpu.sync_copy(rows2_v, acc_sh.at[dst_v.at[j1]], add=True)
            return carry

        lax.fori_loop(0, TILE_CHUNKS // 2, body, 0)
        plsc.subcore_barrier()
        for k in range(3):
            pltpu.sync_copy(acc_sh.at[pl.ds(base + k * ZC, ZC)], stag_v)
            pltpu.sync_copy(stag_v, out_hbm.at[c, pl.ds(base + k * ZC, ZC)])

        @pl.when(s == NTILES - 1)
        def _():
            pltpu.sync_copy(acc_sh.at[pl.ds(NH - 8, 8)], stag_v.at[pl.ds(0, 8)])
            pltpu.sync_copy(stag_v.at[pl.ds(0, 8)], out_hbm.at[c, pl.ds(NH - 8, 8)])

    return agg


_B = 1000


def _tc_prep(deg2, x):
    def body(deg_ref, x_ref, d_ref, g_ref):
        deg = deg_ref[0] + deg_ref[1] + 1.0
        d = lax.rsqrt(deg)
        d_ref[...] = d
        g_ref[...] = x_ref[...] * d

    return pl.pallas_call(
        body,
        grid=(N // _B,),
        in_specs=[
            pl.BlockSpec((2, _B, 1), lambda i: (0, i, 0)),
            pl.BlockSpec((_B, 128), lambda i: (i, 0)),
        ],
        out_specs=[
            pl.BlockSpec((_B, 1), lambda i: (i, 0)),
            pl.BlockSpec((_B, 128), lambda i: (i, 0)),
        ],
        out_shape=[
            jax.ShapeDtypeStruct((N, 1), jnp.float32),
            jax.ShapeDtypeStruct((N, 128), jnp.float32),
        ],
    )(deg2.reshape(2, N, 1), x)


def _tc_layer1(acc1, g1, d, W1, b1):
    def body(a_ref, g_ref, d_ref, W_ref, b_ref, out_ref):
        y = (a_ref[0] + g_ref[...]) * d_ref[...]
        h = jnp.dot(y, W_ref[...], preferred_element_type=jnp.float32) + b_ref[...]
        g = jnp.maximum(h, 0.0) * d_ref[...]
        out_ref[0] = g[:, :128]
        out_ref[1] = g[:, 128:]

    return pl.pallas_call(
        body,
        grid=(N // _B,),
        in_specs=[
            pl.BlockSpec((1, _B, 128), lambda i: (i // 5, i % 5, 0)),
            pl.BlockSpec((_B, 128), lambda i: (i, 0)),
            pl.BlockSpec((_B, 1), lambda i: (i, 0)),
            pl.BlockSpec((128, 256), lambda i: (0, 0)),
            pl.BlockSpec((1, 256), lambda i: (0, 0)),
        ],
        out_specs=pl.BlockSpec((2, _B, 128), lambda i: (0, i, 0)),
        out_shape=jax.ShapeDtypeStruct((2, N, 128), jnp.float32),
    )(acc1, g1, d, W1, b1.reshape(1, 256))


def _tc_layer2(acc2a, acc2b, g2, d, W2, b2, fc_W, fc_b):
    def body(a0_ref, a1_ref, g_ref, d_ref, W_ref, b_ref, fw_ref, fb_ref, out_ref):
        a = jnp.concatenate([a0_ref[0] + g_ref[0], a1_ref[0] + g_ref[1]], axis=1)
        y = a * d_ref[...]
        h = jnp.dot(y, W_ref[...], preferred_element_type=jnp.float32) + b_ref[...]
        h = jnp.maximum(h, 0.0)
        logits = jnp.dot(h, fw_ref[...], preferred_element_type=jnp.float32) + fb_ref[...]
        m = jnp.max(logits, axis=1, keepdims=True)
        lse = jnp.log(jnp.sum(jnp.exp(logits - m), axis=1, keepdims=True)) + m
        out_ref[...] = logits - lse

    return pl.pallas_call(
        body,
        grid=(N // _B,),
        in_specs=[
            pl.BlockSpec((1, _B, 128), lambda i: (i // 5, i % 5, 0)),
            pl.BlockSpec((1, _B, 128), lambda i: (i // 5, i % 5, 0)),
            pl.BlockSpec((2, _B, 128), lambda i: (0, i, 0)),
            pl.BlockSpec((_B, 1), lambda i: (i, 0)),
            pl.BlockSpec((256, 256), lambda i: (0, 0)),
            pl.BlockSpec((1, 256), lambda i: (0, 0)),
            pl.BlockSpec((256, 64), lambda i: (0, 0)),
            pl.BlockSpec((1, 64), lambda i: (0, 0)),
        ],
        out_specs=pl.BlockSpec((_B, 64), lambda i: (i, 0)),
        out_shape=jax.ShapeDtypeStruct((N, 64), jnp.float32),
    )(acc2a, acc2b, g2, d, W2, b2.reshape(1, 256), fc_W, fc_b.reshape(1, 64))


def kernel(x, edge_index, W1, b1, W2, b2, fc_W, fc_b):
    src = edge_index[0]
    dst = edge_index[1]
    pad = EP - E
    src_p = jnp.concatenate([src, jnp.zeros((pad,), jnp.int32)])
    dst_p = jnp.concatenate([dst, jnp.full((pad,), N, jnp.int32)])
    dst32 = dst_p.reshape(32, DEG_CHUNKS, 128)
    src3 = src_p.reshape(NTILES, TILE_CHUNKS, 128)
    nh = N // 2
    sent = nh + (jnp.arange(EP, dtype=jnp.int32) & 127)
    dstns = jnp.stack([
        jnp.where(dst_p < nh, dst_p, sent),
        jnp.where(dst_p >= nh, dst_p - nh, sent),
    ]).reshape(2, NTILES, TILE_CHUNKS, 128)

    deg2 = _make_deg()(dst32)
    d, g1 = _tc_prep(deg2, x)
    agg = _make_agg_ns()
    acc1 = agg(g1, src3, dstns)
    g2 = _tc_layer1(acc1, g1, d, W1, b1)
    acc2a = agg(g2[0], src3, dstns)
    acc2b = agg(g2[1], src3, dstns)
    return _tc_layer2(acc2a, acc2b, g2, d, W2, b2, fc_W, fc_b)
